# baseline (speedup 1.0000x reference)
import functools

import jax
import jax.numpy as jnp
from jax import lax
from jax.experimental import pallas as pl
from jax.experimental.pallas import tpu as pltpu

_KT = 1024
_NBLK = 1024


def _mlp_kernel(x_ref, w0_ref, w1_ref, w2_ref, b0_ref, b1_ref, b2_ref,
                out_ref, acc_ref, *, nk, nn, chunk, rows):
    k = pl.program_id(0)
    n = pl.program_id(1)
    nblk = w0_ref.shape[0]

    xb = x_ref[...].astype(jnp.bfloat16)
    for c in range(0, nblk, chunk):
        w0b = w0_ref[pl.ds(c, chunk), :].astype(jnp.bfloat16)
        part = lax.dot_general(xb, w0b, (((1,), (1,)), ((), ())),
                               preferred_element_type=jnp.float32)
        col = n * nblk + c
        cur = acc_ref[:, pl.ds(col, chunk)]
        acc_ref[:, pl.ds(col, chunk)] = jnp.where(k == 0, part, cur + part)

    @pl.when((k == nk - 1) & (n == nn - 1))
    def _epilogue():
        B = out_ref.shape[0]
        w1 = w1_ref[...]
        w2 = w2_ref[...]

        def body(i, _):
            r = i * rows
            h1 = jnp.maximum(acc_ref[pl.ds(r, rows), :] + b0_ref[...], 0.0)
            h1 = h1.astype(jnp.bfloat16)
            h2 = lax.dot_general(h1, w1, (((1,), (1,)), ((), ())),
                                 preferred_element_type=jnp.float32)
            h2 = jnp.maximum(h2 + b1_ref[...], 0.0).astype(jnp.bfloat16)
            o = lax.dot_general(h2, w2, (((1,), (1,)), ((), ())),
                                preferred_element_type=jnp.float32)
            out_ref[pl.ds(r, rows), :] = o + b2_ref[...]
            return 0

        lax.fori_loop(0, B // rows, body, 0, unroll=False)


def kernel(x, W0, b0, W1, b1, W2, b2):
    B, F0 = x.shape
    F1 = W0.shape[0]
    F2 = W1.shape[0]
    F3 = W2.shape[0]
    kt = min(_KT, F0)
    nk = F0 // kt
    nblk = min(_NBLK, F1)
    nn = F1 // nblk
    chunk = min(1024, nblk)
    rows = min(128, B)

    w1b = W1.astype(jnp.bfloat16)
    w2b = W2.astype(jnp.bfloat16)
    b0r = b0.reshape(1, F1)
    b1r = b1.reshape(1, F2)
    b2r = b2.reshape(1, F3)

    body = functools.partial(_mlp_kernel, nk=nk, nn=nn, chunk=chunk, rows=rows)

    return pl.pallas_call(
        body,
        grid=(nk, nn),
        in_specs=[
            pl.BlockSpec((B, kt), lambda k, n: (0, k)),
            pl.BlockSpec((nblk, kt), lambda k, n: (n, k)),
            pl.BlockSpec((F2, F1), lambda k, n: (0, 0)),
            pl.BlockSpec((F3, F2), lambda k, n: (0, 0)),
            pl.BlockSpec((1, F1), lambda k, n: (0, 0)),
            pl.BlockSpec((1, F2), lambda k, n: (0, 0)),
            pl.BlockSpec((1, F3), lambda k, n: (0, 0)),
        ],
        out_specs=pl.BlockSpec((B, F3), lambda k, n: (0, 0)),
        out_shape=jax.ShapeDtypeStruct((B, F3), jnp.float32),
        scratch_shapes=[pltpu.VMEM((B, F1), jnp.float32)],
        compiler_params=pltpu.CompilerParams(
            dimension_semantics=("arbitrary", "arbitrary"),
        ),
    )(x, W0, w1b, w2b, b0r, b1r, b2r)


# speedup vs baseline: 1.0332x; 1.0332x over previous
import functools

import jax
import jax.numpy as jnp
from jax import lax
from jax.experimental import pallas as pl
from jax.experimental.pallas import tpu as pltpu

_KT = 2048
_NBLK = 1024


def _layer0_kernel(x_ref, w0_ref, h_ref, *, chunk):
    k = pl.program_id(0)
    n = pl.program_id(1)
    nblk = w0_ref.shape[0]

    xb = x_ref[...].astype(jnp.bfloat16)
    for c in range(0, nblk, chunk):
        w0b = w0_ref[pl.ds(c, chunk), :].astype(jnp.bfloat16)
        part = lax.dot_general(xb, w0b, (((1,), (1,)), ((), ())),
                               preferred_element_type=jnp.float32)
        col = n * nblk + c
        cur = h_ref[:, pl.ds(col, chunk)]
        h_ref[:, pl.ds(col, chunk)] = jnp.where(k == 0, part, cur + part)


def _tail_kernel(h_ref, w1_ref, w2_ref, b0_ref, b1_ref, b2_ref, out_ref):
    h1 = jnp.maximum(h_ref[...] + b0_ref[...], 0.0).astype(jnp.bfloat16)
    h2 = lax.dot_general(h1, w1_ref[...], (((1,), (1,)), ((), ())),
                         preferred_element_type=jnp.float32)
    h2 = jnp.maximum(h2 + b1_ref[...], 0.0).astype(jnp.bfloat16)
    o = lax.dot_general(h2, w2_ref[...], (((1,), (1,)), ((), ())),
                        preferred_element_type=jnp.float32)
    out_ref[...] = o + b2_ref[...]


def kernel(x, W0, b0, W1, b1, W2, b2):
    B, F0 = x.shape
    F1 = W0.shape[0]
    F2 = W1.shape[0]
    F3 = W2.shape[0]
    kt = min(_KT, F0)
    nk = F0 // kt
    nblk = min(_NBLK, F1)
    nn = F1 // nblk
    chunk = min(512, nblk)

    h1 = pl.pallas_call(
        functools.partial(_layer0_kernel, chunk=chunk),
        grid=(nk, nn),
        in_specs=[
            pl.BlockSpec((B, kt), lambda k, n: (0, k)),
            pl.BlockSpec((nblk, kt), lambda k, n: (n, k)),
        ],
        out_specs=pl.BlockSpec((B, F1), lambda k, n: (0, 0)),
        out_shape=jax.ShapeDtypeStruct((B, F1), jnp.float32),
        compiler_params=pltpu.CompilerParams(
            dimension_semantics=("arbitrary", "arbitrary"),
        ),
    )(x, W0)

    w1b = W1.astype(jnp.bfloat16)
    w2b = W2.astype(jnp.bfloat16)
    b0r = b0.reshape(1, F1)
    b1r = b1.reshape(1, F2)
    b2r = b2.reshape(1, F3)

    rows = min(128, B)
    return pl.pallas_call(
        _tail_kernel,
        grid=(B // rows,),
        in_specs=[
            pl.BlockSpec((rows, F1), lambda i: (i, 0)),
            pl.BlockSpec((F2, F1), lambda i: (0, 0)),
            pl.BlockSpec((F3, F2), lambda i: (0, 0)),
            pl.BlockSpec((1, F1), lambda i: (0, 0)),
            pl.BlockSpec((1, F2), lambda i: (0, 0)),
            pl.BlockSpec((1, F3), lambda i: (0, 0)),
        ],
        out_specs=pl.BlockSpec((rows, F3), lambda i: (i, 0)),
        out_shape=jax.ShapeDtypeStruct((B, F3), jnp.float32),
        compiler_params=pltpu.CompilerParams(
            dimension_semantics=("arbitrary",),
        ),
    )(h1, w1b, w2b, b0r, b1r, b2r)


# speedup vs baseline: 1.1035x; 1.0681x over previous
import functools

import jax
import jax.numpy as jnp
from jax import lax
from jax.experimental import pallas as pl
from jax.experimental.pallas import tpu as pltpu

_KT = 2048
_NBLK = 1024


def _layer0_kernel(x_ref, w0_ref, h_ref, *, chunk):
    k = pl.program_id(0)
    n = pl.program_id(1)
    nblk = w0_ref.shape[0]

    xb = x_ref[...].astype(jnp.bfloat16)
    for c in range(0, nblk, chunk):
        w0b = w0_ref[pl.ds(c, chunk), :].astype(jnp.bfloat16)
        part = lax.dot_general(xb, w0b, (((1,), (1,)), ((), ())),
                               preferred_element_type=jnp.float32)
        col = n * nblk + c
        cur = h_ref[:, pl.ds(col, chunk)]
        h_ref[:, pl.ds(col, chunk)] = jnp.where(k == 0, part, cur + part)


def _tail_kernel(h_ref, w1_ref, w2_ref, b0_ref, b1_ref, b2_ref, out_ref):
    h1 = jnp.maximum(h_ref[...] + b0_ref[...], 0.0).astype(jnp.bfloat16)
    h2 = lax.dot_general(h1, w1_ref[...], (((1,), (1,)), ((), ())),
                         preferred_element_type=jnp.float32)
    h2 = jnp.maximum(h2 + b1_ref[...], 0.0).astype(jnp.bfloat16)
    o = lax.dot_general(h2, w2_ref[...], (((1,), (1,)), ((), ())),
                        preferred_element_type=jnp.float32)
    out_ref[...] = o + b2_ref[...]


def kernel(x, W0, b0, W1, b1, W2, b2):
    B, F0 = x.shape
    F1 = W0.shape[0]
    F2 = W1.shape[0]
    F3 = W2.shape[0]
    kt = min(_KT, F0)
    nk = F0 // kt
    nblk = min(_NBLK, F1)
    nn = F1 // nblk
    chunk = min(1024, nblk)

    h1 = pl.pallas_call(
        functools.partial(_layer0_kernel, chunk=chunk),
        grid=(nk, nn),
        in_specs=[
            pl.BlockSpec((B, kt), lambda k, n: (0, k)),
            pl.BlockSpec((nblk, kt), lambda k, n: (n, k)),
        ],
        out_specs=pl.BlockSpec((B, F1), lambda k, n: (0, 0)),
        out_shape=jax.ShapeDtypeStruct((B, F1), jnp.float32),
        compiler_params=pltpu.CompilerParams(
            dimension_semantics=("arbitrary", "arbitrary"),
        ),
    )(x, W0)

    w1b = W1.astype(jnp.bfloat16)
    w2b = W2.astype(jnp.bfloat16)
    b0r = b0.reshape(1, F1)
    b1r = b1.reshape(1, F2)
    b2r = b2.reshape(1, F3)

    rows = min(256, B)
    return pl.pallas_call(
        _tail_kernel,
        grid=(B // rows,),
        in_specs=[
            pl.BlockSpec((rows, F1), lambda i: (i, 0)),
            pl.BlockSpec((F2, F1), lambda i: (0, 0)),
            pl.BlockSpec((F3, F2), lambda i: (0, 0)),
            pl.BlockSpec((1, F1), lambda i: (0, 0)),
            pl.BlockSpec((1, F2), lambda i: (0, 0)),
            pl.BlockSpec((1, F3), lambda i: (0, 0)),
        ],
        out_specs=pl.BlockSpec((rows, F3), lambda i: (i, 0)),
        out_shape=jax.ShapeDtypeStruct((B, F3), jnp.float32),
        compiler_params=pltpu.CompilerParams(
            dimension_semantics=("arbitrary",),
        ),
    )(h1, w1b, w2b, b0r, b1r, b2r)
